# baseline (speedup 1.0000x reference)
import jax
import jax.numpy as jnp
from jax import lax
from jax.experimental import pallas as pl
from jax.experimental.pallas import tpu as pltpu
from jax.experimental.pallas import tpu_sc as plsc

_SIZE = 1000000
_NC, _NS, _LANES = 2, 16, 16
_NW = _NC * _NS
_CHUNK = 31256
_LAST_START = _SIZE - _CHUNK


def _sc_body(cache_hbm, idx_hbm, val_hbm, out_hbm, buf_v, idx_v, val_v):
    wid = lax.axis_index("s") * _NC + lax.axis_index("c")
    start = jnp.minimum(wid * _CHUNK, _LAST_START)
    pltpu.sync_copy(idx_hbm, idx_v)
    pltpu.sync_copy(val_hbm, val_v)
    pltpu.sync_copy(cache_hbm.at[pl.ds(start, _CHUNK)], buf_v)
    vidx = idx_v[...]
    vval = val_v[...]
    off = vidx - start
    lane = lax.iota(jnp.int32, _LANES)
    mask = (off >= 0) & (off < _CHUNK) & (lane == 0)
    plsc.store_scatter(buf_v, [off], vval, mask=mask)
    pltpu.sync_copy(buf_v, out_hbm.at[pl.ds(start, _CHUNK)])


def kernel(cache, index, value):
    idx_arr = jnp.full((_LANES,), index, dtype=jnp.int32)
    val_arr = jnp.full((_LANES,), value, dtype=jnp.float32)
    mesh = plsc.VectorSubcoreMesh(
        core_axis_name="c", subcore_axis_name="s",
        num_cores=_NC, num_subcores=_NS,
    )
    f = pl.kernel(
        _sc_body,
        out_type=jax.ShapeDtypeStruct((_SIZE,), jnp.float32),
        mesh=mesh,
        scratch_types=[
            pltpu.VMEM((_CHUNK,), jnp.float32),
            pltpu.VMEM((_LANES,), jnp.int32),
            pltpu.VMEM((_LANES,), jnp.float32),
        ],
        compiler_params=pltpu.CompilerParams(needs_layout_passes=False),
    )
    return f(cache, idx_arr, val_arr)


# speedup vs baseline: 1.0743x; 1.0743x over previous
import jax
import jax.numpy as jnp
from jax import lax
from jax.experimental import pallas as pl
from jax.experimental.pallas import tpu as pltpu
from jax.experimental.pallas import tpu_sc as plsc

_SIZE = 1000000
_NC, _NS, _LANES = 2, 16, 16
_NW = _NC * _NS
_NB = 4
_CHUNK = 31264
_SUB = _CHUNK // _NB
_LAST_START = _SIZE - _CHUNK


def _sc_body(cache_hbm, par_hbm, out_hbm, buf_v, par_v, gsem, ssem):
    wid = lax.axis_index("s") * _NC + lax.axis_index("c")
    start = jnp.minimum(wid * _CHUNK, _LAST_START)
    par_cp = pltpu.async_copy(par_hbm, par_v, ssem)
    gathers = []
    for b in range(_NB):
        gathers.append(pltpu.async_copy(
            cache_hbm.at[pl.ds(start + b * _SUB, _SUB)],
            buf_v.at[pl.ds(b * _SUB, _SUB)], gsem))
    par_cp.wait()
    vidx = par_v[pl.ds(0, _LANES)]
    vval = plsc.bitcast(par_v[pl.ds(_LANES, _LANES)], jnp.float32)
    off = vidx - start
    lane = lax.iota(jnp.int32, _LANES)
    mask = (off >= 0) & (off < _CHUNK) & (lane == 0)
    for g in gathers:
        g.wait()
    plsc.store_scatter(buf_v, [off], vval, mask=mask)
    scatters = []
    for b in range(_NB):
        scatters.append(pltpu.async_copy(
            buf_v.at[pl.ds(b * _SUB, _SUB)],
            out_hbm.at[pl.ds(start + b * _SUB, _SUB)], ssem))
    for s in scatters:
        s.wait()


def kernel(cache, index, value):
    idx_arr = jnp.full((_LANES,), index, dtype=jnp.int32)
    val_arr = jnp.full((_LANES,), value, dtype=jnp.float32)
    par_arr = jnp.concatenate(
        [idx_arr, lax.bitcast_convert_type(val_arr, jnp.int32)])
    mesh = plsc.VectorSubcoreMesh(
        core_axis_name="c", subcore_axis_name="s",
        num_cores=_NC, num_subcores=_NS,
    )
    f = pl.kernel(
        _sc_body,
        out_type=jax.ShapeDtypeStruct((_SIZE,), jnp.float32),
        mesh=mesh,
        scratch_types=[
            pltpu.VMEM((_CHUNK,), jnp.float32),
            pltpu.VMEM((2 * _LANES,), jnp.int32),
            pltpu.SemaphoreType.DMA,
            pltpu.SemaphoreType.DMA,
        ],
        compiler_params=pltpu.CompilerParams(needs_layout_passes=False),
    )
    return f(cache, par_arr)
